# baseline (speedup 1.0000x reference)
import jax
import jax.numpy as jnp
from jax import lax
from jax.experimental import pallas as pl
from jax.experimental.pallas import tpu as pltpu
from jax.experimental.pallas import tpu_sc as plsc

N_KEYS = 50000
N_QUERIES = 512
DIM = 64
DPAD = 128
K = 3
M = 4

BLK = 6272
N_PAD = 50176
N_BLOCKS = N_PAD // BLK
QB = 128
NQB = N_QUERIES // QB
NCH = 2
PADVAL = 1.0e4

BIG = 3.0e38
BIGI = 2**31 - 1

NC = 2
NS = 16
NW = NC * NS
QPW = N_QUERIES // NW


def _select_kernel(q_ref, x_ref, bi_ref, d_scr, sv_scr, si_scr):
    ki = pl.program_id(1)

    @pl.when(ki == 0)
    def _init():
        sv_scr[...] = jnp.full((NCH * M * 8, QB), BIG, jnp.float32)
        si_scr[...] = jnp.zeros((NCH * M * 8, QB), jnp.int32)

    kb = x_ref[...]
    kn = jnp.sum(kb * kb, axis=1)
    dots = lax.dot_general(
        kb, q_ref[...], (((1,), (1,)), ((), ())),
        preferred_element_type=jnp.float32,
        precision=lax.Precision.HIGHEST,
    )
    d_scr[...] = kn[:, None] - 2.0 * dots

    base = ki * BLK
    sub = lax.broadcasted_iota(jnp.int32, (8, QB), 0)

    nslot = NCH * M
    carry = tuple([sv_scr[pl.ds(j * 8, 8), :] for j in range(nslot)]
                  + [si_scr[pl.ds(j * 8, 8), :] for j in range(nslot)])

    def body(i, c):
        bvs = list(c[:nslot])
        bis = list(c[nslot:])
        for u in range(NCH):
            cv = d_scr[pl.ds((i * NCH + u) * 8, 8), :]
            ci = sub + (base + (i * NCH + u) * 8)
            for j in range(u * M, u * M + M):
                take = cv < bvs[j]
                nv = jnp.where(take, cv, bvs[j])
                ni = jnp.where(take, ci, bis[j])
                cv = jnp.where(take, bvs[j], cv)
                ci = jnp.where(take, bis[j], ci)
                bvs[j] = nv
                bis[j] = ni
        return tuple(bvs + bis)

    carry = lax.fori_loop(0, BLK // (8 * NCH), body, carry)
    for j in range(nslot):
        sv_scr[pl.ds(j * 8, 8), :] = carry[j]
        si_scr[pl.ds(j * 8, 8), :] = carry[nslot + j]

    @pl.when(ki == N_BLOCKS - 1)
    def _merge():
        vals = sv_scr[...]
        idxs = si_scr[...]
        rows = []
        for _t in range(M):
            m = jnp.min(vals, axis=0, keepdims=True)
            am = jnp.min(jnp.where(vals == m, idxs, BIGI), axis=0, keepdims=True)
            vals = jnp.where(idxs == am, BIG, vals)
            rows.append(am)
        bi_ref[...] = jnp.concatenate(rows, axis=0)[None]


def _run_select(q2, xpad):
    return pl.pallas_call(
        _select_kernel,
        grid=(NQB, N_BLOCKS),
        in_specs=[
            pl.BlockSpec((QB, DIM), lambda qi, ki: (qi, 0)),
            pl.BlockSpec((BLK, DIM), lambda qi, ki: (ki, 0)),
        ],
        out_specs=[
            pl.BlockSpec((1, M, QB), lambda qi, ki: (qi, 0, 0)),
        ],
        out_shape=[
            jax.ShapeDtypeStruct((NQB, M, QB), jnp.int32),
        ],
        scratch_shapes=[
            pltpu.VMEM((BLK, QB), jnp.float32),
            pltpu.VMEM((NCH * M * 8, QB), jnp.float32),
            pltpu.VMEM((NCH * M * 8, QB), jnp.int32),
        ],
    )(q2, xpad)


def _rescore_body(xp_hbm, yp_hbm, qtb_hbm, cand_hbm, out_hbm, idx_hbm,
                  cand_v, xrows_v, qt_v, yidx_v, yrows_v, outbuf_v, sem):
    wid = lax.axis_index("s") * NC + lax.axis_index("c")
    lane = lax.iota(jnp.int32, 16)

    pltpu.sync_copy(cand_hbm.at[pl.ds(wid * (QPW * M), QPW * M)], cand_v)
    pltpu.async_copy(xp_hbm.at[cand_v], xrows_v, sem).wait()
    pltpu.sync_copy(qtb_hbm.at[pl.ds(wid * (DIM * QPW), DIM * QPW)], qt_v)

    dvals = []
    ivals = []
    for m in range(M):
        rows = lane * M + m
        s = jnp.zeros((16,), jnp.float32)
        comp = jnp.zeros((16,), jnp.float32)
        for dd in range(DIM):
            xg = plsc.load_gather(xrows_v, [rows, jnp.full((16,), dd, jnp.int32)])
            qg = qt_v[pl.ds(dd * 16, 16)]
            df = xg - qg
            t = df * df
            y_ = t - comp
            tmp = s + y_
            comp = (tmp - s) - y_
            s = tmp
        dvals.append(s)
        ivals.append(plsc.load_gather(cand_v, [rows]))

    sel_i = []
    for _r in range(K):
        bd = dvals[0]
        bi = ivals[0]
        for m in range(1, M):
            better = (dvals[m] < bd) | ((dvals[m] == bd) & (ivals[m] < bi))
            bd = jnp.where(better, dvals[m], bd)
            bi = jnp.where(better, ivals[m], bi)
        sel_i.append(bi)
        for m in range(M):
            dvals[m] = jnp.where(ivals[m] == bi, BIG, dvals[m])

    for r in range(K):
        plsc.store_scatter(yidx_v, [lane * K + r], sel_i[r])

    pltpu.async_copy(yp_hbm.at[yidx_v], yrows_v, sem).wait()

    for ql in range(QPW):
        for cb in range(DIM // 16):
            a = yrows_v[ql * K + 0, pl.ds(cb * 16, 16)]
            b = yrows_v[ql * K + 1, pl.ds(cb * 16, 16)]
            c = yrows_v[ql * K + 2, pl.ds(cb * 16, 16)]
            outbuf_v[pl.ds(ql * DIM + cb * 16, 16)] = (a + b + c) / 3.0

    pltpu.sync_copy(outbuf_v, out_hbm.at[pl.ds(wid * (QPW * DIM), QPW * DIM)])
    pltpu.sync_copy(yidx_v, idx_hbm.at[pl.ds(wid * (QPW * K), QPW * K)])


def _run_rescore(xp, yp, qtb, cand_flat):
    mesh = plsc.VectorSubcoreMesh(
        core_axis_name="c", subcore_axis_name="s",
        num_cores=NC, num_subcores=NS,
    )
    return pl.kernel(
        _rescore_body,
        out_type=[
            jax.ShapeDtypeStruct((N_QUERIES * DIM,), jnp.float32),
            jax.ShapeDtypeStruct((N_QUERIES * K,), jnp.int32),
        ],
        mesh=mesh,
        compiler_params=pltpu.CompilerParams(needs_layout_passes=False),
        scratch_types=[
            pltpu.VMEM((QPW * M,), jnp.int32),
            pltpu.VMEM((QPW * M, DPAD), jnp.float32),
            pltpu.VMEM((DIM * QPW,), jnp.float32),
            pltpu.VMEM((QPW * K,), jnp.int32),
            pltpu.VMEM((QPW * K, DPAD), jnp.float32),
            pltpu.VMEM((QPW * DIM,), jnp.float32),
            pltpu.SemaphoreType.DMA,
        ],
    )(xp, yp, qtb, cand_flat)


def kernel(dataTensorX, dataTensorY, inputTensor):
    xf = dataTensorX.reshape(N_KEYS, DIM)
    q2 = inputTensor.reshape(N_QUERIES, DIM)
    xpad = jnp.pad(xf, ((0, N_PAD - N_KEYS), (0, 0)), constant_values=PADVAL)

    (cand3,) = _run_select(q2, xpad)
    cand_flat = cand3.transpose(0, 2, 1).reshape(N_QUERIES * M)

    xp = jnp.pad(xf, ((0, 0), (0, DPAD - DIM)))
    yp = jnp.pad(dataTensorY, ((0, 0), (0, DPAD - DIM)))
    qtb = q2.T.reshape(DIM, NW, QPW).transpose(1, 0, 2).reshape(NW * DIM * QPW)

    out_flat, idx_flat = _run_rescore(xp, yp, qtb, cand_flat)
    return (out_flat.reshape(N_QUERIES, DIM), idx_flat.reshape(N_QUERIES, K))


# speedup vs baseline: 1.1902x; 1.1902x over previous
import jax
import jax.numpy as jnp
from jax import lax
from jax.experimental import pallas as pl
from jax.experimental.pallas import tpu as pltpu
from jax.experimental.pallas import tpu_sc as plsc

N_KEYS = 50000
N_QUERIES = 512
DIM = 64
DPAD = 128
K = 3
M = 4

BLK = 6272
N_PAD = 50176
N_BLOCKS = N_PAD // BLK
QB = 128
NQB = N_QUERIES // QB
NCH = 1
UNROLL = 8
PADVAL = 1.0e4

BIG = 3.0e38
BIGI = 2**31 - 1

NC = 2
NS = 16
NW = NC * NS
QPW = N_QUERIES // NW


def _select_kernel(q_ref, x_ref, bi_ref, d_scr, sv_scr, si_scr):
    ki = pl.program_id(1)

    @pl.when(ki == 0)
    def _init():
        sv_scr[...] = jnp.full((NCH * M * 8, QB), BIG, jnp.float32)
        si_scr[...] = jnp.zeros((NCH * M * 8, QB), jnp.int32)

    kb = x_ref[...]
    kn = jnp.sum(kb * kb, axis=1)
    dots = lax.dot_general(
        kb, q_ref[...], (((1,), (1,)), ((), ())),
        preferred_element_type=jnp.float32,
        precision=lax.Precision.HIGHEST,
    )
    d_scr[...] = kn[:, None] - 2.0 * dots

    base = ki * BLK
    sub = lax.broadcasted_iota(jnp.int32, (8, QB), 0)

    nslot = NCH * M
    carry = tuple([sv_scr[pl.ds(j * 8, 8), :] for j in range(nslot)]
                  + [si_scr[pl.ds(j * 8, 8), :] for j in range(nslot)])

    def body(i, c):
        bvs = list(c[:nslot])
        bis = list(c[nslot:])
        for u in range(UNROLL):
            ch = u % NCH
            cv = d_scr[pl.ds((i * UNROLL + u) * 8, 8), :]
            ci = sub + (base + (i * UNROLL + u) * 8)
            for j in range(ch * M, ch * M + M):
                take = cv < bvs[j]
                nv = jnp.where(take, cv, bvs[j])
                ni = jnp.where(take, ci, bis[j])
                cv = jnp.where(take, bvs[j], cv)
                ci = jnp.where(take, bis[j], ci)
                bvs[j] = nv
                bis[j] = ni
        return tuple(bvs + bis)

    carry = lax.fori_loop(0, BLK // (8 * UNROLL), body, carry)
    for j in range(nslot):
        sv_scr[pl.ds(j * 8, 8), :] = carry[j]
        si_scr[pl.ds(j * 8, 8), :] = carry[nslot + j]

    @pl.when(ki == N_BLOCKS - 1)
    def _merge():
        vals = sv_scr[...]
        idxs = si_scr[...]
        rows = []
        for _t in range(M):
            m = jnp.min(vals, axis=0, keepdims=True)
            am = jnp.min(jnp.where(vals == m, idxs, BIGI), axis=0, keepdims=True)
            vals = jnp.where(idxs == am, BIG, vals)
            rows.append(am)
        bi_ref[...] = jnp.concatenate(rows, axis=0)[None]


def _run_select(q2, xpad):
    return pl.pallas_call(
        _select_kernel,
        grid=(NQB, N_BLOCKS),
        in_specs=[
            pl.BlockSpec((QB, DIM), lambda qi, ki: (qi, 0)),
            pl.BlockSpec((BLK, DIM), lambda qi, ki: (ki, 0)),
        ],
        out_specs=[
            pl.BlockSpec((1, M, QB), lambda qi, ki: (qi, 0, 0)),
        ],
        out_shape=[
            jax.ShapeDtypeStruct((NQB, M, QB), jnp.int32),
        ],
        scratch_shapes=[
            pltpu.VMEM((BLK, QB), jnp.float32),
            pltpu.VMEM((NCH * M * 8, QB), jnp.float32),
            pltpu.VMEM((NCH * M * 8, QB), jnp.int32),
        ],
    )(q2, xpad)


def _rescore_body(xp_hbm, yp_hbm, qtb_hbm, cand_hbm, out_hbm, idx_hbm,
                  cand_v, xrows_v, qt_v, yidx_v, yrows_v, outbuf_v, sem):
    wid = lax.axis_index("s") * NC + lax.axis_index("c")
    lane = lax.iota(jnp.int32, 16)

    pltpu.sync_copy(cand_hbm.at[pl.ds(wid * (QPW * M), QPW * M)], cand_v)
    pltpu.async_copy(xp_hbm.at[cand_v], xrows_v, sem).wait()
    pltpu.sync_copy(qtb_hbm.at[pl.ds(wid * (DIM * QPW), DIM * QPW)], qt_v)

    dvals = []
    ivals = []
    for m in range(M):
        rows = lane * M + m
        s = jnp.zeros((16,), jnp.float32)
        comp = jnp.zeros((16,), jnp.float32)
        for dd in range(DIM):
            xg = plsc.load_gather(xrows_v, [rows, jnp.full((16,), dd, jnp.int32)])
            qg = qt_v[pl.ds(dd * 16, 16)]
            df = xg - qg
            t = df * df
            y_ = t - comp
            tmp = s + y_
            comp = (tmp - s) - y_
            s = tmp
        dvals.append(s)
        ivals.append(plsc.load_gather(cand_v, [rows]))

    sel_i = []
    for _r in range(K):
        bd = dvals[0]
        bi = ivals[0]
        for m in range(1, M):
            better = (dvals[m] < bd) | ((dvals[m] == bd) & (ivals[m] < bi))
            bd = jnp.where(better, dvals[m], bd)
            bi = jnp.where(better, ivals[m], bi)
        sel_i.append(bi)
        for m in range(M):
            dvals[m] = jnp.where(ivals[m] == bi, BIG, dvals[m])

    for r in range(K):
        plsc.store_scatter(yidx_v, [lane * K + r], sel_i[r])

    pltpu.async_copy(yp_hbm.at[yidx_v], yrows_v, sem).wait()

    for ql in range(QPW):
        for cb in range(DIM // 16):
            a = yrows_v[ql * K + 0, pl.ds(cb * 16, 16)]
            b = yrows_v[ql * K + 1, pl.ds(cb * 16, 16)]
            c = yrows_v[ql * K + 2, pl.ds(cb * 16, 16)]
            outbuf_v[pl.ds(ql * DIM + cb * 16, 16)] = (a + b + c) / 3.0

    pltpu.sync_copy(outbuf_v, out_hbm.at[pl.ds(wid * (QPW * DIM), QPW * DIM)])
    pltpu.sync_copy(yidx_v, idx_hbm.at[pl.ds(wid * (QPW * K), QPW * K)])


def _run_rescore(xp, yp, qtb, cand_flat):
    mesh = plsc.VectorSubcoreMesh(
        core_axis_name="c", subcore_axis_name="s",
        num_cores=NC, num_subcores=NS,
    )
    return pl.kernel(
        _rescore_body,
        out_type=[
            jax.ShapeDtypeStruct((N_QUERIES * DIM,), jnp.float32),
            jax.ShapeDtypeStruct((N_QUERIES * K,), jnp.int32),
        ],
        mesh=mesh,
        compiler_params=pltpu.CompilerParams(needs_layout_passes=False),
        scratch_types=[
            pltpu.VMEM((QPW * M,), jnp.int32),
            pltpu.VMEM((QPW * M, DPAD), jnp.float32),
            pltpu.VMEM((DIM * QPW,), jnp.float32),
            pltpu.VMEM((QPW * K,), jnp.int32),
            pltpu.VMEM((QPW * K, DPAD), jnp.float32),
            pltpu.VMEM((QPW * DIM,), jnp.float32),
            pltpu.SemaphoreType.DMA,
        ],
    )(xp, yp, qtb, cand_flat)


def kernel(dataTensorX, dataTensorY, inputTensor):
    xf = dataTensorX.reshape(N_KEYS, DIM)
    q2 = inputTensor.reshape(N_QUERIES, DIM)
    xpad = jnp.pad(xf, ((0, N_PAD - N_KEYS), (0, 0)), constant_values=PADVAL)

    (cand3,) = _run_select(q2, xpad)
    cand_flat = cand3.transpose(0, 2, 1).reshape(N_QUERIES * M)

    xp = jnp.pad(xf, ((0, 0), (0, DPAD - DIM)))
    yp = jnp.pad(dataTensorY, ((0, 0), (0, DPAD - DIM)))
    qtb = q2.T.reshape(DIM, NW, QPW).transpose(1, 0, 2).reshape(NW * DIM * QPW)

    out_flat, idx_flat = _run_rescore(xp, yp, qtb, cand_flat)
    return (out_flat.reshape(N_QUERIES, DIM), idx_flat.reshape(N_QUERIES, K))


# speedup vs baseline: 1.2737x; 1.0701x over previous
import jax
import jax.numpy as jnp
from jax import lax
from jax.experimental import pallas as pl
from jax.experimental.pallas import tpu as pltpu
from jax.experimental.pallas import tpu_sc as plsc

N_KEYS = 50000
N_QUERIES = 512
DIM = 64
DPAD = 128
K = 3
M = 4

BLK = 6272
N_PAD = 50176
N_BLOCKS = N_PAD // BLK
QB = 256
NQB = N_QUERIES // QB
NCH = 1
UNROLL = 8
PADVAL = 1.0e4

BIG = 3.0e38
BIGI = 2**31 - 1

NC = 2
NS = 16
NW = NC * NS
QPW = N_QUERIES // NW


def _select_kernel(q_ref, x_ref, bi_ref, d_scr, sv_scr, si_scr):
    ki = pl.program_id(1)

    @pl.when(ki == 0)
    def _init():
        sv_scr[...] = jnp.full((NCH * M * 8, QB), BIG, jnp.float32)
        si_scr[...] = jnp.zeros((NCH * M * 8, QB), jnp.int32)

    kb = x_ref[...]
    kn = jnp.sum(kb * kb, axis=1)
    dots = lax.dot_general(
        kb, q_ref[...], (((1,), (1,)), ((), ())),
        preferred_element_type=jnp.float32,
        precision=lax.Precision.HIGHEST,
    )
    d_scr[...] = kn[:, None] - 2.0 * dots

    base = ki * BLK
    sub = lax.broadcasted_iota(jnp.int32, (8, QB), 0)

    nslot = NCH * M
    carry = tuple([sv_scr[pl.ds(j * 8, 8), :] for j in range(nslot)]
                  + [si_scr[pl.ds(j * 8, 8), :] for j in range(nslot)])

    def body(i, c):
        bvs = list(c[:nslot])
        bis = list(c[nslot:])
        for u in range(UNROLL):
            ch = u % NCH
            cv = d_scr[pl.ds((i * UNROLL + u) * 8, 8), :]
            ci = sub + (base + (i * UNROLL + u) * 8)
            for j in range(ch * M, ch * M + M):
                take = cv < bvs[j]
                nv = jnp.where(take, cv, bvs[j])
                ni = jnp.where(take, ci, bis[j])
                cv = jnp.where(take, bvs[j], cv)
                ci = jnp.where(take, bis[j], ci)
                bvs[j] = nv
                bis[j] = ni
        return tuple(bvs + bis)

    carry = lax.fori_loop(0, BLK // (8 * UNROLL), body, carry)
    for j in range(nslot):
        sv_scr[pl.ds(j * 8, 8), :] = carry[j]
        si_scr[pl.ds(j * 8, 8), :] = carry[nslot + j]

    @pl.when(ki == N_BLOCKS - 1)
    def _merge():
        vals = sv_scr[...]
        idxs = si_scr[...]
        rows = []
        for _t in range(M):
            m = jnp.min(vals, axis=0, keepdims=True)
            am = jnp.min(jnp.where(vals == m, idxs, BIGI), axis=0, keepdims=True)
            vals = jnp.where(idxs == am, BIG, vals)
            rows.append(am)
        bi_ref[...] = jnp.concatenate(rows, axis=0)[None]


def _run_select(q2, xpad):
    return pl.pallas_call(
        _select_kernel,
        grid=(NQB, N_BLOCKS),
        in_specs=[
            pl.BlockSpec((QB, DIM), lambda qi, ki: (qi, 0)),
            pl.BlockSpec((BLK, DIM), lambda qi, ki: (ki, 0)),
        ],
        out_specs=[
            pl.BlockSpec((1, M, QB), lambda qi, ki: (qi, 0, 0)),
        ],
        out_shape=[
            jax.ShapeDtypeStruct((NQB, M, QB), jnp.int32),
        ],
        scratch_shapes=[
            pltpu.VMEM((BLK, QB), jnp.float32),
            pltpu.VMEM((NCH * M * 8, QB), jnp.float32),
            pltpu.VMEM((NCH * M * 8, QB), jnp.int32),
        ],
    )(q2, xpad)


def _rescore_body(xp_hbm, yp_hbm, qtb_hbm, cand_hbm, out_hbm, idx_hbm,
                  cand_v, xrows_v, qt_v, yidx_v, yrows_v, outbuf_v, sem):
    wid = lax.axis_index("s") * NC + lax.axis_index("c")
    lane = lax.iota(jnp.int32, 16)

    pltpu.sync_copy(cand_hbm.at[pl.ds(wid * (QPW * M), QPW * M)], cand_v)
    pltpu.async_copy(xp_hbm.at[cand_v], xrows_v, sem).wait()
    pltpu.sync_copy(qtb_hbm.at[pl.ds(wid * (DIM * QPW), DIM * QPW)], qt_v)

    dvals = []
    ivals = []
    for m in range(M):
        rows = lane * M + m
        s = jnp.zeros((16,), jnp.float32)
        comp = jnp.zeros((16,), jnp.float32)
        for dd in range(DIM):
            xg = plsc.load_gather(xrows_v, [rows, jnp.full((16,), dd, jnp.int32)])
            qg = qt_v[pl.ds(dd * 16, 16)]
            df = xg - qg
            t = df * df
            y_ = t - comp
            tmp = s + y_
            comp = (tmp - s) - y_
            s = tmp
        dvals.append(s)
        ivals.append(plsc.load_gather(cand_v, [rows]))

    sel_i = []
    for _r in range(K):
        bd = dvals[0]
        bi = ivals[0]
        for m in range(1, M):
            better = (dvals[m] < bd) | ((dvals[m] == bd) & (ivals[m] < bi))
            bd = jnp.where(better, dvals[m], bd)
            bi = jnp.where(better, ivals[m], bi)
        sel_i.append(bi)
        for m in range(M):
            dvals[m] = jnp.where(ivals[m] == bi, BIG, dvals[m])

    for r in range(K):
        plsc.store_scatter(yidx_v, [lane * K + r], sel_i[r])

    pltpu.async_copy(yp_hbm.at[yidx_v], yrows_v, sem).wait()

    for ql in range(QPW):
        for cb in range(DIM // 16):
            a = yrows_v[ql * K + 0, pl.ds(cb * 16, 16)]
            b = yrows_v[ql * K + 1, pl.ds(cb * 16, 16)]
            c = yrows_v[ql * K + 2, pl.ds(cb * 16, 16)]
            outbuf_v[pl.ds(ql * DIM + cb * 16, 16)] = (a + b + c) / 3.0

    pltpu.sync_copy(outbuf_v, out_hbm.at[pl.ds(wid * (QPW * DIM), QPW * DIM)])
    pltpu.sync_copy(yidx_v, idx_hbm.at[pl.ds(wid * (QPW * K), QPW * K)])


def _run_rescore(xp, yp, qtb, cand_flat):
    mesh = plsc.VectorSubcoreMesh(
        core_axis_name="c", subcore_axis_name="s",
        num_cores=NC, num_subcores=NS,
    )
    return pl.kernel(
        _rescore_body,
        out_type=[
            jax.ShapeDtypeStruct((N_QUERIES * DIM,), jnp.float32),
            jax.ShapeDtypeStruct((N_QUERIES * K,), jnp.int32),
        ],
        mesh=mesh,
        compiler_params=pltpu.CompilerParams(needs_layout_passes=False),
        scratch_types=[
            pltpu.VMEM((QPW * M,), jnp.int32),
            pltpu.VMEM((QPW * M, DPAD), jnp.float32),
            pltpu.VMEM((DIM * QPW,), jnp.float32),
            pltpu.VMEM((QPW * K,), jnp.int32),
            pltpu.VMEM((QPW * K, DPAD), jnp.float32),
            pltpu.VMEM((QPW * DIM,), jnp.float32),
            pltpu.SemaphoreType.DMA,
        ],
    )(xp, yp, qtb, cand_flat)


def kernel(dataTensorX, dataTensorY, inputTensor):
    xf = dataTensorX.reshape(N_KEYS, DIM)
    q2 = inputTensor.reshape(N_QUERIES, DIM)
    xpad = jnp.pad(xf, ((0, N_PAD - N_KEYS), (0, 0)), constant_values=PADVAL)

    (cand3,) = _run_select(q2, xpad)
    cand_flat = cand3.transpose(0, 2, 1).reshape(N_QUERIES * M)

    xp = jnp.pad(xf, ((0, 0), (0, DPAD - DIM)))
    yp = jnp.pad(dataTensorY, ((0, 0), (0, DPAD - DIM)))
    qtb = q2.T.reshape(DIM, NW, QPW).transpose(1, 0, 2).reshape(NW * DIM * QPW)

    out_flat, idx_flat = _run_rescore(xp, yp, qtb, cand_flat)
    return (out_flat.reshape(N_QUERIES, DIM), idx_flat.reshape(N_QUERIES, K))


# speedup vs baseline: 1.3022x; 1.0223x over previous
import jax
import jax.numpy as jnp
from jax import lax
from jax.experimental import pallas as pl
from jax.experimental.pallas import tpu as pltpu
from jax.experimental.pallas import tpu_sc as plsc

N_KEYS = 50000
N_QUERIES = 512
DIM = 64
DPAD = 128
K = 3
M = 4

BLK = 6272
N_PAD = 50176
N_BLOCKS = N_PAD // BLK
QB = 512
NQB = N_QUERIES // QB
NCH = 1
UNROLL = 8
PADVAL = 1.0e4

BIG = 3.0e38
BIGI = 2**31 - 1

NC = 2
NS = 16
NW = NC * NS
QPW = N_QUERIES // NW


def _select_kernel(q_ref, x_ref, bi_ref, d_scr, sv_scr, si_scr):
    ki = pl.program_id(1)

    @pl.when(ki == 0)
    def _init():
        sv_scr[...] = jnp.full((NCH * M * 8, QB), BIG, jnp.float32)
        si_scr[...] = jnp.zeros((NCH * M * 8, QB), jnp.int32)

    kb = x_ref[...]
    kn = jnp.sum(kb * kb, axis=1)
    dots = lax.dot_general(
        kb, q_ref[...], (((1,), (1,)), ((), ())),
        preferred_element_type=jnp.float32,
        precision=lax.Precision.HIGHEST,
    )
    d_scr[...] = kn[:, None] - 2.0 * dots

    base = ki * BLK
    sub = lax.broadcasted_iota(jnp.int32, (8, QB), 0)

    nslot = NCH * M
    carry = tuple([sv_scr[pl.ds(j * 8, 8), :] for j in range(nslot)]
                  + [si_scr[pl.ds(j * 8, 8), :] for j in range(nslot)])

    def body(i, c):
        bvs = list(c[:nslot])
        bis = list(c[nslot:])
        for u in range(UNROLL):
            ch = u % NCH
            cv = d_scr[pl.ds((i * UNROLL + u) * 8, 8), :]
            ci = sub + (base + (i * UNROLL + u) * 8)
            for j in range(ch * M, ch * M + M):
                take = cv < bvs[j]
                nv = jnp.where(take, cv, bvs[j])
                ni = jnp.where(take, ci, bis[j])
                cv = jnp.where(take, bvs[j], cv)
                ci = jnp.where(take, bis[j], ci)
                bvs[j] = nv
                bis[j] = ni
        return tuple(bvs + bis)

    carry = lax.fori_loop(0, BLK // (8 * UNROLL), body, carry)
    for j in range(nslot):
        sv_scr[pl.ds(j * 8, 8), :] = carry[j]
        si_scr[pl.ds(j * 8, 8), :] = carry[nslot + j]

    @pl.when(ki == N_BLOCKS - 1)
    def _merge():
        vals = sv_scr[...]
        idxs = si_scr[...]
        rows = []
        for _t in range(M):
            m = jnp.min(vals, axis=0, keepdims=True)
            am = jnp.min(jnp.where(vals == m, idxs, BIGI), axis=0, keepdims=True)
            vals = jnp.where(idxs == am, BIG, vals)
            rows.append(am)
        bi_ref[...] = jnp.concatenate(rows, axis=0)[None]


def _run_select(q2, xpad):
    return pl.pallas_call(
        _select_kernel,
        grid=(NQB, N_BLOCKS),
        in_specs=[
            pl.BlockSpec((QB, DIM), lambda qi, ki: (qi, 0)),
            pl.BlockSpec((BLK, DIM), lambda qi, ki: (ki, 0)),
        ],
        out_specs=[
            pl.BlockSpec((1, M, QB), lambda qi, ki: (qi, 0, 0)),
        ],
        out_shape=[
            jax.ShapeDtypeStruct((NQB, M, QB), jnp.int32),
        ],
        scratch_shapes=[
            pltpu.VMEM((BLK, QB), jnp.float32),
            pltpu.VMEM((NCH * M * 8, QB), jnp.float32),
            pltpu.VMEM((NCH * M * 8, QB), jnp.int32),
        ],
    )(q2, xpad)


def _rescore_body(xp_hbm, yp_hbm, qtb_hbm, cand_hbm, out_hbm, idx_hbm,
                  cand_v, xrows_v, qt_v, yidx_v, yrows_v, outbuf_v, sem):
    wid = lax.axis_index("s") * NC + lax.axis_index("c")
    lane = lax.iota(jnp.int32, 16)

    pltpu.sync_copy(cand_hbm.at[pl.ds(wid * (QPW * M), QPW * M)], cand_v)
    pltpu.async_copy(xp_hbm.at[cand_v], xrows_v, sem).wait()
    pltpu.sync_copy(qtb_hbm.at[pl.ds(wid * (DIM * QPW), DIM * QPW)], qt_v)

    dvals = []
    ivals = []
    for m in range(M):
        rows = lane * M + m
        s = jnp.zeros((16,), jnp.float32)
        comp = jnp.zeros((16,), jnp.float32)
        for dd in range(DIM):
            xg = plsc.load_gather(xrows_v, [rows, jnp.full((16,), dd, jnp.int32)])
            qg = qt_v[pl.ds(dd * 16, 16)]
            df = xg - qg
            t = df * df
            y_ = t - comp
            tmp = s + y_
            comp = (tmp - s) - y_
            s = tmp
        dvals.append(s)
        ivals.append(plsc.load_gather(cand_v, [rows]))

    sel_i = []
    for _r in range(K):
        bd = dvals[0]
        bi = ivals[0]
        for m in range(1, M):
            better = (dvals[m] < bd) | ((dvals[m] == bd) & (ivals[m] < bi))
            bd = jnp.where(better, dvals[m], bd)
            bi = jnp.where(better, ivals[m], bi)
        sel_i.append(bi)
        for m in range(M):
            dvals[m] = jnp.where(ivals[m] == bi, BIG, dvals[m])

    for r in range(K):
        plsc.store_scatter(yidx_v, [lane * K + r], sel_i[r])

    pltpu.async_copy(yp_hbm.at[yidx_v], yrows_v, sem).wait()

    for ql in range(QPW):
        for cb in range(DIM // 16):
            a = yrows_v[ql * K + 0, pl.ds(cb * 16, 16)]
            b = yrows_v[ql * K + 1, pl.ds(cb * 16, 16)]
            c = yrows_v[ql * K + 2, pl.ds(cb * 16, 16)]
            outbuf_v[pl.ds(ql * DIM + cb * 16, 16)] = (a + b + c) / 3.0

    pltpu.sync_copy(outbuf_v, out_hbm.at[pl.ds(wid * (QPW * DIM), QPW * DIM)])
    pltpu.sync_copy(yidx_v, idx_hbm.at[pl.ds(wid * (QPW * K), QPW * K)])


def _run_rescore(xp, yp, qtb, cand_flat):
    mesh = plsc.VectorSubcoreMesh(
        core_axis_name="c", subcore_axis_name="s",
        num_cores=NC, num_subcores=NS,
    )
    return pl.kernel(
        _rescore_body,
        out_type=[
            jax.ShapeDtypeStruct((N_QUERIES * DIM,), jnp.float32),
            jax.ShapeDtypeStruct((N_QUERIES * K,), jnp.int32),
        ],
        mesh=mesh,
        compiler_params=pltpu.CompilerParams(needs_layout_passes=False),
        scratch_types=[
            pltpu.VMEM((QPW * M,), jnp.int32),
            pltpu.VMEM((QPW * M, DPAD), jnp.float32),
            pltpu.VMEM((DIM * QPW,), jnp.float32),
            pltpu.VMEM((QPW * K,), jnp.int32),
            pltpu.VMEM((QPW * K, DPAD), jnp.float32),
            pltpu.VMEM((QPW * DIM,), jnp.float32),
            pltpu.SemaphoreType.DMA,
        ],
    )(xp, yp, qtb, cand_flat)


def kernel(dataTensorX, dataTensorY, inputTensor):
    xf = dataTensorX.reshape(N_KEYS, DIM)
    q2 = inputTensor.reshape(N_QUERIES, DIM)
    xpad = jnp.pad(xf, ((0, N_PAD - N_KEYS), (0, 0)), constant_values=PADVAL)

    (cand3,) = _run_select(q2, xpad)
    cand_flat = cand3.transpose(0, 2, 1).reshape(N_QUERIES * M)

    xp = jnp.pad(xf, ((0, 0), (0, DPAD - DIM)))
    yp = jnp.pad(dataTensorY, ((0, 0), (0, DPAD - DIM)))
    qtb = q2.T.reshape(DIM, NW, QPW).transpose(1, 0, 2).reshape(NW * DIM * QPW)

    out_flat, idx_flat = _run_rescore(xp, yp, qtb, cand_flat)
    return (out_flat.reshape(N_QUERIES, DIM), idx_flat.reshape(N_QUERIES, K))


# speedup vs baseline: 1.5286x; 1.1739x over previous
import jax
import jax.numpy as jnp
from jax import lax
from jax.experimental import pallas as pl
from jax.experimental.pallas import tpu as pltpu
from jax.experimental.pallas import tpu_sc as plsc

N_KEYS = 50000
N_QUERIES = 512
DIM = 64
DPAD = 128
K = 3
M = 4

BLK = 6272
N_PAD = 50176
N_BLOCKS = N_PAD // BLK
QB = 512
NQB = N_QUERIES // QB
NCH = 1
UNROLL = 8
PADVAL = 1.0e4

BIG = 3.0e38
BIGI = 2**31 - 1

NC = 2
NS = 16
NW = NC * NS
QPW = N_QUERIES // NW


def _select_kernel(q_ref, x_ref, bi_ref, d_scr, sv_scr, si_scr):
    ki = pl.program_id(1)

    @pl.when(ki == 0)
    def _init():
        sv_scr[...] = jnp.full((NCH * M * 8, QB), BIG, jnp.float32)
        si_scr[...] = jnp.zeros((NCH * M * 8, QB), jnp.int32)

    kb = x_ref[...]
    kn = jnp.sum(kb * kb, axis=1)
    qf = q_ref[...]
    qhi = qf.astype(jnp.bfloat16)
    qlo = (qf - qhi.astype(jnp.float32)).astype(jnp.bfloat16)
    khi = kb.astype(jnp.bfloat16)
    klo = (kb - khi.astype(jnp.float32)).astype(jnp.bfloat16)
    dn = (((1,), (1,)), ((), ()))
    dots = lax.dot_general(khi, qhi, dn, preferred_element_type=jnp.float32)
    dots += lax.dot_general(khi, qlo, dn, preferred_element_type=jnp.float32)
    dots += lax.dot_general(klo, qhi, dn, preferred_element_type=jnp.float32)
    d_scr[...] = kn[:, None] - 2.0 * dots

    base = ki * BLK
    sub = lax.broadcasted_iota(jnp.int32, (8, QB), 0)

    nslot = NCH * M
    carry = tuple([sv_scr[pl.ds(j * 8, 8), :] for j in range(nslot)]
                  + [si_scr[pl.ds(j * 8, 8), :] for j in range(nslot)])

    def body(i, c):
        bvs = list(c[:nslot])
        bis = list(c[nslot:])
        for u in range(UNROLL):
            ch = u % NCH
            cv = d_scr[pl.ds((i * UNROLL + u) * 8, 8), :]
            ci = sub + (base + (i * UNROLL + u) * 8)
            for j in range(ch * M, ch * M + M):
                take = cv < bvs[j]
                nv = jnp.where(take, cv, bvs[j])
                ni = jnp.where(take, ci, bis[j])
                cv = jnp.where(take, bvs[j], cv)
                ci = jnp.where(take, bis[j], ci)
                bvs[j] = nv
                bis[j] = ni
        return tuple(bvs + bis)

    carry = lax.fori_loop(0, BLK // (8 * UNROLL), body, carry)
    for j in range(nslot):
        sv_scr[pl.ds(j * 8, 8), :] = carry[j]
        si_scr[pl.ds(j * 8, 8), :] = carry[nslot + j]

    @pl.when(ki == N_BLOCKS - 1)
    def _merge():
        vals = sv_scr[...]
        idxs = si_scr[...]
        rows = []
        for _t in range(M):
            m = jnp.min(vals, axis=0, keepdims=True)
            am = jnp.min(jnp.where(vals == m, idxs, BIGI), axis=0, keepdims=True)
            vals = jnp.where(idxs == am, BIG, vals)
            rows.append(am)
        bi_ref[...] = jnp.concatenate(rows, axis=0)[None]


def _run_select(q2, xpad):
    return pl.pallas_call(
        _select_kernel,
        grid=(NQB, N_BLOCKS),
        in_specs=[
            pl.BlockSpec((QB, DIM), lambda qi, ki: (qi, 0)),
            pl.BlockSpec((BLK, DIM), lambda qi, ki: (ki, 0)),
        ],
        out_specs=[
            pl.BlockSpec((1, M, QB), lambda qi, ki: (qi, 0, 0)),
        ],
        out_shape=[
            jax.ShapeDtypeStruct((NQB, M, QB), jnp.int32),
        ],
        scratch_shapes=[
            pltpu.VMEM((BLK, QB), jnp.float32),
            pltpu.VMEM((NCH * M * 8, QB), jnp.float32),
            pltpu.VMEM((NCH * M * 8, QB), jnp.int32),
        ],
    )(q2, xpad)


def _rescore_body(xp_hbm, yp_hbm, qtb_hbm, cand_hbm, out_hbm, idx_hbm,
                  cand_v, xrows_v, qt_v, yidx_v, yrows_v, outbuf_v, sem):
    wid = lax.axis_index("s") * NC + lax.axis_index("c")
    lane = lax.iota(jnp.int32, 16)

    pltpu.sync_copy(cand_hbm.at[pl.ds(wid * (QPW * M), QPW * M)], cand_v)
    pltpu.async_copy(xp_hbm.at[cand_v], xrows_v, sem).wait()
    pltpu.sync_copy(qtb_hbm.at[pl.ds(wid * (DIM * QPW), DIM * QPW)], qt_v)

    dvals = []
    ivals = []
    for m in range(M):
        rows = lane * M + m
        s = jnp.zeros((16,), jnp.float32)
        comp = jnp.zeros((16,), jnp.float32)
        for dd in range(DIM):
            xg = plsc.load_gather(xrows_v, [rows, jnp.full((16,), dd, jnp.int32)])
            qg = qt_v[pl.ds(dd * 16, 16)]
            df = xg - qg
            t = df * df
            y_ = t - comp
            tmp = s + y_
            comp = (tmp - s) - y_
            s = tmp
        dvals.append(s)
        ivals.append(plsc.load_gather(cand_v, [rows]))

    sel_i = []
    for _r in range(K):
        bd = dvals[0]
        bi = ivals[0]
        for m in range(1, M):
            better = (dvals[m] < bd) | ((dvals[m] == bd) & (ivals[m] < bi))
            bd = jnp.where(better, dvals[m], bd)
            bi = jnp.where(better, ivals[m], bi)
        sel_i.append(bi)
        for m in range(M):
            dvals[m] = jnp.where(ivals[m] == bi, BIG, dvals[m])

    for r in range(K):
        plsc.store_scatter(yidx_v, [lane * K + r], sel_i[r])

    pltpu.async_copy(yp_hbm.at[yidx_v], yrows_v, sem).wait()

    for ql in range(QPW):
        for cb in range(DIM // 16):
            a = yrows_v[ql * K + 0, pl.ds(cb * 16, 16)]
            b = yrows_v[ql * K + 1, pl.ds(cb * 16, 16)]
            c = yrows_v[ql * K + 2, pl.ds(cb * 16, 16)]
            outbuf_v[pl.ds(ql * DIM + cb * 16, 16)] = (a + b + c) / 3.0

    pltpu.sync_copy(outbuf_v, out_hbm.at[pl.ds(wid * (QPW * DIM), QPW * DIM)])
    pltpu.sync_copy(yidx_v, idx_hbm.at[pl.ds(wid * (QPW * K), QPW * K)])


def _run_rescore(xp, yp, qtb, cand_flat):
    mesh = plsc.VectorSubcoreMesh(
        core_axis_name="c", subcore_axis_name="s",
        num_cores=NC, num_subcores=NS,
    )
    return pl.kernel(
        _rescore_body,
        out_type=[
            jax.ShapeDtypeStruct((N_QUERIES * DIM,), jnp.float32),
            jax.ShapeDtypeStruct((N_QUERIES * K,), jnp.int32),
        ],
        mesh=mesh,
        compiler_params=pltpu.CompilerParams(needs_layout_passes=False),
        scratch_types=[
            pltpu.VMEM((QPW * M,), jnp.int32),
            pltpu.VMEM((QPW * M, DPAD), jnp.float32),
            pltpu.VMEM((DIM * QPW,), jnp.float32),
            pltpu.VMEM((QPW * K,), jnp.int32),
            pltpu.VMEM((QPW * K, DPAD), jnp.float32),
            pltpu.VMEM((QPW * DIM,), jnp.float32),
            pltpu.SemaphoreType.DMA,
        ],
    )(xp, yp, qtb, cand_flat)


def kernel(dataTensorX, dataTensorY, inputTensor):
    xf = dataTensorX.reshape(N_KEYS, DIM)
    q2 = inputTensor.reshape(N_QUERIES, DIM)
    xpad = jnp.pad(xf, ((0, N_PAD - N_KEYS), (0, 0)), constant_values=PADVAL)

    (cand3,) = _run_select(q2, xpad)
    cand_flat = cand3.transpose(0, 2, 1).reshape(N_QUERIES * M)

    xp = jnp.pad(xf, ((0, 0), (0, DPAD - DIM)))
    yp = jnp.pad(dataTensorY, ((0, 0), (0, DPAD - DIM)))
    qtb = q2.T.reshape(DIM, NW, QPW).transpose(1, 0, 2).reshape(NW * DIM * QPW)

    out_flat, idx_flat = _run_rescore(xp, yp, qtb, cand_flat)
    return (out_flat.reshape(N_QUERIES, DIM), idx_flat.reshape(N_QUERIES, K))
